# Initial kernel scaffold; baseline (speedup 1.0000x reference)
import jax
import jax.numpy as jnp
from jax.experimental import pallas as pl


def kernel(f_node, f_edge, edges, soe, conn_W, conn_b, norm_g, norm_b, cu_W, cu_b, cu_ng, cu_nb, sym_W, sym_b, sym_ng, sym_nb, tri_W, tri_b, tri_ng, tri_nb, sk_fg, sk_fb, sk_sW, sk_sb, sk_eW, sk_eb, sk_ug, sk_ub, ff1_W, ff2_W, ffn_g, ffn_b):
    raise NotImplementedError("write your pallas kernel here")


# speedup vs baseline: 4.4367x; 4.4367x over previous
import functools

import jax
import jax.numpy as jnp
from jax import lax
from jax.experimental import pallas as pl
from jax.experimental.pallas import tpu as pltpu
from jax.experimental.pallas import tpu_sc as plsc

L_N, E_N, T_N, D_N = 10000, 160000, 100000, 128
TP = 100352
SENT = 2 ** 30

CH = 10000
NROUND = 8
CPT = 1000
SEG_PER_TILE = TP // 16
NBMAX = SEG_PER_TILE // 128


def _ln(x, g, b):
    m = jnp.mean(x, axis=-1, keepdims=True)
    v = jnp.mean((x - m) ** 2, axis=-1, keepdims=True)
    return (x - m) / jnp.sqrt(v + 1e-5) * g + b


def _gelu(x):
    return 0.5 * x * (1.0 + lax.erf(x * 0.7071067811865476))


def _pq_body(fn, wsrc, wdst, p_o, q_o):
    x = fn[...]
    p_o[...] = jnp.dot(x, wsrc[...], preferred_element_type=jnp.float32)
    q_o[...] = jnp.dot(x, wdst[...], preferred_element_type=jnp.float32)


def _pq_call(f_node, wsrc, wdst):
    blk = 2000
    return pl.pallas_call(
        _pq_body,
        grid=(L_N // blk,),
        in_specs=[
            pl.BlockSpec((blk, D_N), lambda i: (i, 0)),
            pl.BlockSpec((D_N, D_N), lambda i: (0, 0)),
            pl.BlockSpec((D_N, D_N), lambda i: (0, 0)),
        ],
        out_specs=[
            pl.BlockSpec((blk, D_N), lambda i: (i, 0)),
            pl.BlockSpec((blk, D_N), lambda i: (i, 0)),
        ],
        out_shape=[
            jax.ShapeDtypeStruct((L_N, D_N), jnp.float32),
            jax.ShapeDtypeStruct((L_N, D_N), jnp.float32),
        ],
    )(f_node, wsrc, wdst)


_PQ_CHUNK = 200


def _gather_pq_body(p_hbm, q_hbm, src_hbm, dst_hbm, hp_hbm, hq_hbm,
                    idx_v, rows_v, sem):
    c = lax.axis_index("c")
    s = lax.axis_index("s")
    wid = c * 16 + s
    base0 = wid * (E_N // 32)
    for k in range(E_N // 32 // _PQ_CHUNK):
        base = base0 + k * _PQ_CHUNK
        pltpu.sync_copy(src_hbm.at[pl.ds(base, _PQ_CHUNK)], idx_v)
        pltpu.async_copy(p_hbm.at[idx_v], rows_v, sem).wait()
        pltpu.sync_copy(rows_v, hp_hbm.at[pl.ds(base, _PQ_CHUNK)])
        pltpu.sync_copy(dst_hbm.at[pl.ds(base, _PQ_CHUNK)], idx_v)
        pltpu.async_copy(q_hbm.at[idx_v], rows_v, sem).wait()
        pltpu.sync_copy(rows_v, hq_hbm.at[pl.ds(base, _PQ_CHUNK)])


def _gather_pq(p, q, src, dst):
    mesh = plsc.VectorSubcoreMesh(core_axis_name="c", subcore_axis_name="s")
    f = pl.kernel(
        _gather_pq_body,
        out_type=[
            jax.ShapeDtypeStruct((E_N, D_N), jnp.float32),
            jax.ShapeDtypeStruct((E_N, D_N), jnp.float32),
        ],
        mesh=mesh,
        scratch_types=[
            pltpu.VMEM((_PQ_CHUNK,), jnp.int32),
            pltpu.VMEM((_PQ_CHUNK, D_N), jnp.float32),
            pltpu.SemaphoreType.DMA,
        ],
    )
    return f(p, q, src, dst)


_EB = 800
_ENB = E_N // _EB
_EHALF = _ENB // 2


def _stage1_body(hp, hq, fe, hp2, hq2, fe2,
                 wedge, cb, ng, nb2,
                 cuw, cub, cung, cunb,
                 s0w, s1w, symb, syng, synb,
                 t0w, t1w, t2w,
                 u0_o, u1_o, g0_o, g1_o, g2_o):
    def mkfec(a, b, f):
        x = jnp.dot(f[...], wedge[...], preferred_element_type=jnp.float32)
        return _ln(x + a[...] + b[...] + cb[...], ng[...], nb2[...])

    fec_i = mkfec(hp, hq, fe)
    fec_j = mkfec(hp2, hq2, fe2)
    u0 = jnp.dot(fec_i, cuw[...], preferred_element_type=jnp.float32) + cub[...]
    u0_o[...] = _ln(_gelu(u0), cung[...], cunb[...])
    u1 = (jnp.dot(fec_i, s0w[...], preferred_element_type=jnp.float32)
          + jnp.dot(fec_j, s1w[...], preferred_element_type=jnp.float32)
          + symb[...])
    u1_o[...] = _ln(_gelu(u1), syng[...], synb[...])
    g0_o[...] = jnp.dot(fec_i, t0w[...], preferred_element_type=jnp.float32)
    g1_o[...] = jnp.dot(fec_i, t1w[...], preferred_element_type=jnp.float32)
    g2_o[...] = jnp.dot(fec_i, t2w[...], preferred_element_type=jnp.float32)


def _stage1_call(hp, hq, f_edge, wedge, cb, ng, nb2, cuw, cub, cung, cunb,
                 s0w, s1w, symb, syng, synb, t0w, t1w, t2w):
    eb = pl.BlockSpec((_EB, D_N), lambda i: (i, 0))
    ebj = pl.BlockSpec((_EB, D_N), lambda i: ((i + _EHALF) % _ENB, 0))
    w = pl.BlockSpec((D_N, D_N), lambda i: (0, 0))
    v = pl.BlockSpec((1, D_N), lambda i: (0, 0))
    return pl.pallas_call(
        _stage1_body,
        grid=(_ENB,),
        in_specs=[eb, eb, eb, ebj, ebj, ebj,
                  w, v, v, v,
                  w, v, v, v,
                  w, w, v, v, v,
                  w, w, w],
        out_specs=[eb, eb, eb, eb, eb],
        out_shape=[jax.ShapeDtypeStruct((E_N, D_N), jnp.float32)] * 5,
    )(hp, hq, f_edge, hp, hq, f_edge,
      wedge, cb, ng, nb2, cuw, cub, cung, cunb,
      s0w, s1w, symb, syng, synb, t0w, t1w, t2w)


_TRI_CHUNK = 448


def _gather_tri_body(g0_hbm, g1_hbm, g2_hbm, s0_hbm, s1_hbm, s2_hbm,
                     a0_hbm, a1_hbm, a2_hbm, idx_v, rows_v, sem):
    c = lax.axis_index("c")
    s = lax.axis_index("s")
    wid = c * 16 + s
    base0 = wid * (TP // 32)
    srcs = [(s0_hbm, g0_hbm, a0_hbm), (s1_hbm, g1_hbm, a1_hbm),
            (s2_hbm, g2_hbm, a2_hbm)]
    for k in range(TP // 32 // _TRI_CHUNK):
        base = base0 + k * _TRI_CHUNK
        for (si, gi, ai) in srcs:
            pltpu.sync_copy(si.at[pl.ds(base, _TRI_CHUNK)], idx_v)
            pltpu.async_copy(gi.at[idx_v], rows_v, sem).wait()
            pltpu.sync_copy(rows_v, ai.at[pl.ds(base, _TRI_CHUNK)])


def _gather_tri(g0, g1, g2, s0, s1, s2):
    mesh = plsc.VectorSubcoreMesh(core_axis_name="c", subcore_axis_name="s")
    f = pl.kernel(
        _gather_tri_body,
        out_type=[jax.ShapeDtypeStruct((TP, D_N), jnp.float32)] * 3,
        mesh=mesh,
        scratch_types=[
            pltpu.VMEM((_TRI_CHUNK,), jnp.int32),
            pltpu.VMEM((_TRI_CHUNK, D_N), jnp.float32),
            pltpu.SemaphoreType.DMA,
        ],
    )
    return f(g0, g1, g2, s0, s1, s2)


def _trigelu_body(a0, a1, a2, tb, ft_o):
    ft_o[...] = _gelu(a0[...] + a1[...] + a2[...] + tb[...])


def _trigelu_call(a0, a1, a2, tb):
    blk = 3136
    eb = pl.BlockSpec((blk, D_N), lambda i: (i, 0))
    v = pl.BlockSpec((1, D_N), lambda i: (0, 0))
    return pl.pallas_call(
        _trigelu_body,
        grid=(TP // blk,),
        in_specs=[eb, eb, eb, v],
        out_specs=eb,
        out_shape=jax.ShapeDtypeStruct((TP, D_N), jnp.float32),
    )(a0, a1, a2, tb)


def _make_segmean_round(r):
    def body(ft_hbm, seg_hbm, z128_hbm, z16_hbm, ones_hbm,
             sums_hbm, cntp_hbm,
             seg_v, idx_row, rows_v, ones_v, acc, accc, sem):
        c = lax.axis_index("c")
        s = lax.axis_index("s")
        pltpu.sync_copy(seg_hbm.at[pl.ds(s * SEG_PER_TILE, SEG_PER_TILE)],
                        seg_v)
        pltpu.sync_copy(ones_hbm, ones_v)
        base_tri = s * SEG_PER_TILE
        lo = (c * NROUND + r) * CH

        @pl.when(s < 10)
        def _():
            off = pl.multiple_of(s * CPT, 8)
            pltpu.sync_copy(z128_hbm.at[pl.ds(0, CPT)],
                            acc.at[pl.ds(off, CPT)])
            pltpu.sync_copy(z16_hbm.at[pl.ds(0, CPT)],
                            accc.at[pl.ds(off, CPT)])
        plsc.subcore_barrier()

        def batch_body(b, carry2):
            for j in range(8):
                seg16 = seg_v[pl.ds(b * 128 + j * 16, 16)]
                m = (seg16 >= lo) & (seg16 < lo + CH)
                dummy = CH + j * 16 + lax.iota(jnp.int32, 16)
                idx_row[0, pl.ds(j * 16, 16)] = jnp.where(m, seg16 - lo,
                                                          dummy)
            pltpu.sync_copy(ft_hbm.at[pl.ds(base_tri + b * 128, 128)], rows_v)
            pltpu.sync_copy(rows_v, acc.at[idx_row.at[0]], add=True)
            pltpu.sync_copy(ones_v, accc.at[idx_row.at[0]], add=True)
            return carry2
        lax.fori_loop(0, NBMAX, batch_body, 0)
        plsc.subcore_barrier()

        @pl.when(s < 10)
        def _():
            off = pl.multiple_of(s * CPT, 8)
            pltpu.sync_copy(acc.at[pl.ds(off, CPT)],
                            sums_hbm.at[c].at[pl.ds(off, CPT)])
            pltpu.sync_copy(accc.at[pl.ds(off, CPT)],
                            cntp_hbm.at[c].at[pl.ds(off, CPT)])
        plsc.subcore_barrier()
    return body


def _segment_mean(ft, segp, z128, z16, ones16):
    valid = segp < E_N
    segc = jnp.where(valid, segp, E_N)
    sums = jax.ops.segment_sum(jnp.where(valid[:, None], ft, 0.0), segc,
                               num_segments=E_N + 1)[:E_N]
    cnt = jax.ops.segment_sum(valid.astype(jnp.float32), segc,
                              num_segments=E_N + 1)[:E_N]
    cntp = jnp.zeros((E_N, 16), jnp.float32).at[:, 0].set(cnt)
    return sums, cntp


def _segment_mean_sc(ft, segp, z128, z16, ones16):
    mesh = plsc.VectorSubcoreMesh(core_axis_name="c", subcore_axis_name="s")
    sums_parts = []
    cnt_parts = []
    for r in range(NROUND):
        f = pl.kernel(
            _make_segmean_round(r),
            out_type=[
                jax.ShapeDtypeStruct((2, CH, D_N), jnp.float32),
                jax.ShapeDtypeStruct((2, CH, 16), jnp.float32),
            ],
            mesh=mesh,
            scratch_types=[
                pltpu.VMEM((SEG_PER_TILE,), jnp.int32),
                pltpu.VMEM((1, 128), jnp.int32),
                pltpu.VMEM((128, D_N), jnp.float32),
                pltpu.VMEM((128, 16), jnp.float32),
                pltpu.VMEM_SHARED((CH + 128, D_N), jnp.float32),
                pltpu.VMEM_SHARED((CH + 128, 16), jnp.float32),
                pltpu.SemaphoreType.DMA,
            ],
            compiler_params=pltpu.CompilerParams(needs_layout_passes=False),
        )
        srt, crt = f(ft, segp, z128, z16, ones16)
        sums_parts.append(srt)
        cnt_parts.append(crt)
    sums = jnp.concatenate(
        [p[0] for p in sums_parts] + [p[1] for p in sums_parts], axis=0)
    cntp = jnp.concatenate(
        [p[0] for p in cnt_parts] + [p[1] for p in cnt_parts], axis=0)
    return sums, cntp


def _stage2_body(u0, u1, sums, cntp, fe,
                 tg, tb2, fg2, fb2, sw, sb, ew, eb2, ug, ub,
                 ff1, ff2, ng2, nb3, out):
    cnt = jnp.clip(cntp[...][:, 0:1], 1.0, None)
    u2 = _ln(sums[...] / cnt, tg[...], tb2[...])
    u0v = u0[...]
    u1v = u1[...]
    ssum = u0v + u1v + u2
    a = jnp.dot(_ln(ssum, fg2[...], fb2[...]), sw[...],
                preferred_element_type=jnp.float32) + sb[...]
    a = jnp.dot(_gelu(a), ew[...], preferred_element_type=jnp.float32) + eb2[...]
    a0 = a[:, 0:D_N]
    a1 = a[:, D_N:2 * D_N]
    a2 = a[:, 2 * D_N:3 * D_N]
    mx = jnp.maximum(jnp.maximum(a0, a1), a2)
    e0 = jnp.exp(a0 - mx)
    e1 = jnp.exp(a1 - mx)
    e2 = jnp.exp(a2 - mx)
    den = e0 + e1 + e2
    f = _ln(fe[...] + (u0v * e0 + u1v * e1 + u2 * e2) / den, ug[...], ub[...])
    vg = jnp.dot(f, ff1[...], preferred_element_type=jnp.float32)
    val = vg[:, 0:2 * D_N]
    gate = vg[:, 2 * D_N:4 * D_N]
    h = jnp.dot(_gelu(gate) * val, ff2[...], preferred_element_type=jnp.float32)
    out[...] = _ln(f + h, ng2[...], nb3[...])


def _stage2_call(u0, u1, sums, cntp, f_edge,
                 tg, tb2, fg2, fb2, sw, sb, ew, eb2, ug, ub, ff1, ff2,
                 ng2, nb3):
    eb = pl.BlockSpec((_EB, D_N), lambda i: (i, 0))
    cb = pl.BlockSpec((_EB, 16), lambda i: (i, 0))
    v = pl.BlockSpec((1, D_N), lambda i: (0, 0))
    return pl.pallas_call(
        _stage2_body,
        grid=(_ENB,),
        in_specs=[eb, eb, eb, cb, eb,
                  v, v, v, v,
                  pl.BlockSpec((D_N, D_N // 4), lambda i: (0, 0)),
                  pl.BlockSpec((1, D_N // 4), lambda i: (0, 0)),
                  pl.BlockSpec((D_N // 4, 3 * D_N), lambda i: (0, 0)),
                  pl.BlockSpec((1, 3 * D_N), lambda i: (0, 0)),
                  v, v,
                  pl.BlockSpec((D_N, 4 * D_N), lambda i: (0, 0)),
                  pl.BlockSpec((2 * D_N, D_N), lambda i: (0, 0)),
                  v, v],
        out_specs=eb,
        out_shape=jax.ShapeDtypeStruct((E_N, D_N), jnp.float32),
    )(u0, u1, sums, cntp, f_edge,
      tg, tb2, fg2, fb2, sw, sb, ew, eb2, ug, ub, ff1, ff2, ng2, nb3)


def kernel(f_node, f_edge, edges, soe,
           conn_W, conn_b, norm_g, norm_b,
           cu_W, cu_b, cu_ng, cu_nb,
           sym_W, sym_b, sym_ng, sym_nb,
           tri_W, tri_b, tri_ng, tri_nb,
           sk_fg, sk_fb, sk_sW, sk_sb, sk_eW, sk_eb, sk_ug, sk_ub,
           ff1_W, ff2_W, ffn_g, ffn_b):
    D = D_N
    r2 = lambda x: x.reshape(1, -1)

    wsrc = conn_W[:, 0:D].T
    wedge = conn_W[:, D:2 * D].T
    wdst = conn_W[:, 2 * D:3 * D].T
    s0w = sym_W[:, 0:D].T
    s1w = sym_W[:, D:2 * D].T
    t0w = tri_W[:, 0:D].T
    t1w = tri_W[:, D:2 * D].T
    t2w = tri_W[:, 2 * D:3 * D].T

    src = edges[0].astype(jnp.int32)
    dst = edges[1].astype(jnp.int32)
    soe32 = soe.astype(jnp.int32)
    padi = jnp.zeros((TP - T_N,), jnp.int32)
    s0 = jnp.concatenate([soe32[:, 0], padi])
    s1 = jnp.concatenate([soe32[:, 1], padi])
    s2 = jnp.concatenate([soe32[:, 2], padi])
    segp = jnp.concatenate([soe32[:, 2],
                            jnp.full((TP - T_N,), SENT, jnp.int32)])

    z128 = jnp.zeros((CPT, D), jnp.float32)
    z16 = jnp.zeros((CPT, 16), jnp.float32)
    ones16 = jnp.zeros((128, 16), jnp.float32).at[:, 0].set(1.0)

    p, q = _pq_call(f_node, wsrc, wdst)
    hp, hq = _gather_pq(p, q, src, dst)
    u0, u1, g0, g1, g2 = _stage1_call(
        hp, hq, f_edge, wedge, r2(conn_b), r2(norm_g), r2(norm_b),
        cu_W.T, r2(cu_b), r2(cu_ng), r2(cu_nb),
        s0w, s1w, r2(sym_b), r2(sym_ng), r2(sym_nb),
        t0w, t1w, t2w)
    a0, a1, a2 = _gather_tri(g0, g1, g2, s0, s1, s2)
    ft = _trigelu_call(a0, a1, a2, r2(tri_b))
    sums, cntp = _segment_mean(ft, segp, z128, z16, ones16)
    out = _stage2_call(
        u0, u1, sums, cntp, f_edge,
        r2(tri_ng), r2(tri_nb), r2(sk_fg), r2(sk_fb),
        sk_sW.T, r2(sk_sb), sk_eW.T, r2(sk_eb), r2(sk_ug), r2(sk_ub),
        ff1_W.T, ff2_W.T, r2(ffn_g), r2(ffn_b))
    return out
